# Initial kernel scaffold; baseline (speedup 1.0000x reference)
import jax
import jax.numpy as jnp
from jax.experimental import pallas as pl


def kernel(lr, coord, cell, enc_w, enc_b, cls_w1, cls_b1, cls_w2, cls_b2, lt_w1, lt_b1, lt_w2, lt_b2, hv_w1, hv_b1, hv_w2, hv_b2, hv_w3, hv_b3):
    raise NotImplementedError("write your pallas kernel here")


# speedup vs baseline: 16.7394x; 16.7394x over previous
import functools

import jax
import jax.numpy as jnp
from jax import lax
from jax.experimental import pallas as pl
from jax.experimental.pallas import tpu as pltpu
from jax.experimental.pallas import tpu_sc as plsc

F32 = jnp.float32
BF16 = jnp.bfloat16
I32 = jnp.int32

NC, NS, L = 2, 16, 16
NW = NC * NS


def _conv_body(x_ref, w_ref, b_ref, o_ref):
    s_out = o_ref.shape[0]
    cols = [x_ref[pl.ds(dy * 130 + dx, s_out), :]
            for dy in range(3) for dx in range(3)]
    x = jnp.concatenate(cols, axis=1)
    y = jnp.dot(x, w_ref[...], preferred_element_type=F32) + b_ref[...]
    y = jnp.maximum(y, 0.0)
    o_ref[...] = jnp.concatenate([y, x, jnp.zeros((s_out, 37), F32)], axis=1)


def _conv_table(xstrips, wr, b, B, nstrip, s_in, s_out, D):
    return pl.pallas_call(
        _conv_body,
        grid=(B, nstrip),
        in_specs=[
            pl.BlockSpec((None, None, s_in, 3), lambda b_, s_: (b_, s_, 0, 0)),
            pl.BlockSpec((27, D), lambda b_, s_: (0, 0)),
            pl.BlockSpec((1, D), lambda b_, s_: (0, 0)),
        ],
        out_specs=pl.BlockSpec((None, s_out, 128), lambda b_, s_: (b_, s_, 0)),
        out_shape=jax.ShapeDtypeStruct((B, nstrip * s_out, 128), F32),
    )(xstrips, wr, b)


def _sc_gather(cx, cy, ftab, N, f_stride, q_per_b):
    chunk = N // NW
    nsub = chunk // L
    njc = chunk // 128

    mesh = plsc.VectorSubcoreMesh(core_axis_name="c", subcore_axis_name="s",
                                  num_cores=NC, num_subcores=NS)

    @functools.partial(
        pl.kernel, mesh=mesh,
        out_type=jax.ShapeDtypeStruct((N, 128), F32),
        scratch_types=[
            pltpu.VMEM((chunk,), F32),
            pltpu.VMEM((chunk,), F32),
            pltpu.VMEM((chunk,), I32),
            pltpu.VMEM((128, 128), F32),
            pltpu.SemaphoreType.DMA,
        ],
    )
    def k(cx_h, cy_h, ftab_h, fs_h, cxv, cyv, niv, fbuf, sem):
        wid = lax.axis_index("s") * NC + lax.axis_index("c")
        base = wid * chunk
        pltpu.sync_copy(cx_h.at[pl.ds(base, chunk)], cxv)
        pltpu.sync_copy(cy_h.at[pl.ds(base, chunk)], cyv)
        b = base // q_per_b
        f_off = b * f_stride

        def idx_body(i, _):
            s = i * L
            cx16 = cxv[pl.ds(s, L)]
            cy16 = cyv[pl.ds(s, L)]
            fx = (cx16 + 1.0) * 64.0 - 0.5
            fy = (cy16 + 1.0) * 64.0 - 0.5
            gx = jnp.maximum(fx + 0.5, 0.0)
            gy = jnp.maximum(fy + 0.5, 0.0)
            xi = jnp.minimum(gx.astype(I32), 127)
            yi = jnp.minimum(gy.astype(I32), 127)
            niv[pl.ds(s, L)] = f_off + yi * 130 + xi
            return 0

        lax.fori_loop(0, nsub, idx_body, 0)

        def g_body(j, _):
            s = j * 128
            cp = pltpu.make_async_copy(
                ftab_h.at[niv.at[pl.ds(s, 128)]], fbuf, sem)
            cp.start()
            cp.wait()
            pltpu.sync_copy(fbuf, fs_h.at[pl.ds(base + s, 128)])
            return 0

        lax.fori_loop(0, njc, g_body, 0)

    return k(cx, cy, ftab)


def _mlp_body(fs_ref, co_ref, ce_ref,
              cw1, cb1, cw2, cb2, lw1, lb1, lw2, lb2,
              hw1, hb1, hw2, hb2, hw3, hb3,
              pred_ref, diff_ref):
    fb = fs_ref[...]
    f = fb[:, 0:64]
    inp = jnp.concatenate([f, ce_ref[...]], axis=1)

    h = jnp.maximum(jnp.dot(inp, cw1[...], preferred_element_type=F32)
                    + cb1[...], 0.0)
    d = jnp.dot(h, cw2[...], preferred_element_type=F32) + cb2[...]
    m = jnp.max(d, axis=1, keepdims=True)
    e = jnp.exp(d - m)
    diff = e / jnp.sum(e, axis=1, keepdims=True)

    hl = jnp.maximum(jnp.dot(inp, lw1[...], preferred_element_type=F32)
                     + lb1[...], 0.0)
    light = jnp.dot(hl, lw2[...], preferred_element_type=F32) + lb2[...]

    hh = jnp.maximum(jnp.dot(inp, hw1[...], preferred_element_type=F32)
                     + hb1[...], 0.0)
    hh = jnp.maximum(jnp.dot(hh, hw2[...], preferred_element_type=F32)
                     + hb2[...], 0.0)
    heavy = jnp.dot(hh, hw3[...], preferred_element_type=F32) + hb3[...]

    cx = co_ref[:, 1:2]
    cy = co_ref[:, 0:1]
    fx = (cx + 1.0) * 64.0 - 0.5
    fy = (cy + 1.0) * 64.0 - 0.5
    xi = jnp.minimum(jnp.maximum(fx + 0.5, 0.0).astype(I32), 127)
    yi = jnp.minimum(jnp.maximum(fy + 0.5, 0.0).astype(I32), 127)
    x0 = jnp.minimum(jnp.maximum(fx, 0.0).astype(I32), 127)
    y0 = jnp.minimum(jnp.maximum(fy, 0.0).astype(I32), 127)
    wx = fx - jnp.floor(fx)
    wy = fy - jnp.floor(fy)

    nb = fb[:, 64:91]

    def tap(oy, ox):
        c0 = ((oy + 1) * 3 + (ox + 1)) * 3
        return nb[:, c0:c0 + 3]

    ym = y0 < yi
    yp = jnp.logical_and(y0 == yi, y0 < 127)
    xm = x0 < xi
    xp = jnp.logical_and(x0 == xi, x0 < 127)

    def pick(rc, t1, t0):
        return jnp.where(rc, t1, t0)

    v00 = pick(ym, pick(xm, tap(-1, -1), tap(-1, 0)),
               pick(xm, tap(0, -1), tap(0, 0)))
    v01 = pick(ym, pick(xp, tap(-1, 1), tap(-1, 0)),
               pick(xp, tap(0, 1), tap(0, 0)))
    v10 = pick(yp, pick(xm, tap(1, -1), tap(1, 0)),
               pick(xm, tap(0, -1), tap(0, 0)))
    v11 = pick(yp, pick(xp, tap(1, 1), tap(1, 0)),
               pick(xp, tap(0, 1), tap(0, 0)))

    bil = (v00 * (1.0 - wx) * (1.0 - wy) + v01 * wx * (1.0 - wy)
           + v10 * (1.0 - wx) * wy + v11 * wx * wy)

    pred_ref[...] = diff[:, 0:1] * light + diff[:, 1:2] * heavy + bil
    diff_ref[...] = diff


def _mlp_call(fs, coordr, cellr, wts, N, bk):
    grid = (N // bk,)
    row = lambda i: (i, 0)
    cst = lambda i: (0, 0)
    w_specs = [pl.BlockSpec(w.shape, cst) for w in wts]
    return pl.pallas_call(
        _mlp_body,
        grid=grid,
        in_specs=[
            pl.BlockSpec((bk, 128), row),
            pl.BlockSpec((bk, 2), row),
            pl.BlockSpec((bk, 2), row),
        ] + w_specs,
        out_specs=[pl.BlockSpec((bk, 3), row), pl.BlockSpec((bk, 2), row)],
        out_shape=[jax.ShapeDtypeStruct((N, 3), F32),
                   jax.ShapeDtypeStruct((N, 2), F32)],
    )(fs, coordr, cellr, *wts)


def kernel(lr, coord, cell, enc_w, enc_b, cls_w1, cls_b1, cls_w2, cls_b2,
           lt_w1, lt_b1, lt_w2, lt_b2, hv_w1, hv_b1, hv_w2, hv_b2,
           hv_w3, hv_b3):
    B, C, H, W = lr.shape
    _, Hq, Wq, _ = coord.shape
    D = enc_w.shape[0]
    N = B * Hq * Wq
    q_per_b = Hq * Wq

    lrh = jnp.transpose(lr, (0, 2, 3, 1))
    xpad = jnp.pad(lrh, ((0, 0), (1, 1), (1, 1), (0, 0)))
    xflat = xpad.reshape(B, 130 * 130, 3)
    xflat = jnp.pad(xflat, ((0, 0), (0, 4), (0, 0)))
    nstrip, s_out = 8, 2080
    s_in = 2344
    xstrips = jnp.stack(
        [xflat[:, s * s_out: s * s_out + s_in] for s in range(nstrip)], axis=1)
    wr = jnp.transpose(enc_w, (2, 3, 1, 0)).reshape(27, D)
    f_stride = nstrip * s_out

    ftab = _conv_table(xstrips, wr, enc_b.reshape(1, D),
                       B, nstrip, s_in, s_out, D)
    ftab = ftab.reshape(B * f_stride, 128)

    cx = coord[..., 1].reshape(N)
    cy = coord[..., 0].reshape(N)

    fs = _sc_gather(cx, cy, ftab, N, f_stride, q_per_b)

    wts = (
        cls_w1.T, cls_b1.reshape(1, -1), cls_w2.T, cls_b2.reshape(1, -1),
        lt_w1.T, lt_b1.reshape(1, -1), lt_w2.T, lt_b2.reshape(1, -1),
        hv_w1.T, hv_b1.reshape(1, -1), hv_w2.T, hv_b2.reshape(1, -1),
        hv_w3.T, hv_b3.reshape(1, -1),
    )
    predr, diffr = _mlp_call(fs, coord.reshape(N, 2), cell.reshape(N, 2),
                             wts, N, bk=2048)

    pred = predr.reshape(B, Hq, Wq, 3).transpose(0, 3, 1, 2)
    diff = diffr.reshape(B, Hq, Wq, 2).transpose(0, 3, 1, 2)
    return (pred, diff)


# speedup vs baseline: 50.9330x; 3.0427x over previous
import functools

import jax
import jax.numpy as jnp
from jax import lax
from jax.experimental import pallas as pl
from jax.experimental.pallas import tpu as pltpu
from jax.experimental.pallas import tpu_sc as plsc

F32 = jnp.float32
BF16 = jnp.bfloat16
I32 = jnp.int32

NC, NS, L = 2, 16, 16
NW = NC * NS


def _conv_body(x_ref, w_ref, b_ref, o_ref):
    s_out = o_ref.shape[0]
    cols = [x_ref[pl.ds(dy * 130 + dx, s_out), :]
            for dy in range(3) for dx in range(3)]
    x = jnp.concatenate(cols, axis=1)
    y = jnp.dot(x, w_ref[...], preferred_element_type=F32) + b_ref[...]
    y = jnp.maximum(y, 0.0)
    o_ref[...] = jnp.concatenate([y, x, jnp.zeros((s_out, 37), F32)], axis=1)


def _conv_table(xstrips, wr, b, B, nstrip, s_in, s_out, D):
    return pl.pallas_call(
        _conv_body,
        grid=(B, nstrip),
        in_specs=[
            pl.BlockSpec((None, None, s_in, 3), lambda b_, s_: (b_, s_, 0, 0)),
            pl.BlockSpec((27, D), lambda b_, s_: (0, 0)),
            pl.BlockSpec((1, D), lambda b_, s_: (0, 0)),
        ],
        out_specs=pl.BlockSpec((None, s_out, 128), lambda b_, s_: (b_, s_, 0)),
        out_shape=jax.ShapeDtypeStruct((B, nstrip * s_out, 128), F32),
    )(xstrips, wr, b)


def _sc_gather(cx, cy, ftab, N, f_stride, q_per_b):
    chunk = N // NW
    nsub = chunk // L
    njc = chunk // 128

    mesh = plsc.VectorSubcoreMesh(core_axis_name="c", subcore_axis_name="s",
                                  num_cores=NC, num_subcores=NS)

    @functools.partial(
        pl.kernel, mesh=mesh,
        out_type=jax.ShapeDtypeStruct((N, 128), F32),
        scratch_types=[
            pltpu.VMEM((chunk,), F32),
            pltpu.VMEM((chunk,), F32),
            pltpu.VMEM((chunk,), I32),
            pltpu.VMEM((128, 128), F32),
            pltpu.SemaphoreType.DMA,
        ],
    )
    def k(cx_h, cy_h, ftab_h, fs_h, cxv, cyv, niv, fbuf, sem):
        wid = lax.axis_index("s") * NC + lax.axis_index("c")
        base = wid * chunk
        pltpu.sync_copy(cx_h.at[pl.ds(base, chunk)], cxv)
        pltpu.sync_copy(cy_h.at[pl.ds(base, chunk)], cyv)
        b = base // q_per_b
        f_off = b * f_stride

        def idx_body(i, _):
            s = i * L
            cx16 = cxv[pl.ds(s, L)]
            cy16 = cyv[pl.ds(s, L)]
            fx = (cx16 + 1.0) * 64.0 - 0.5
            fy = (cy16 + 1.0) * 64.0 - 0.5
            gx = jnp.maximum(fx + 0.5, 0.0)
            gy = jnp.maximum(fy + 0.5, 0.0)
            xi = jnp.minimum(gx.astype(I32), 127)
            yi = jnp.minimum(gy.astype(I32), 127)
            niv[pl.ds(s, L)] = f_off + yi * 130 + xi
            return 0

        lax.fori_loop(0, nsub, idx_body, 0)

        def g_body(j, _):
            s = j * 128
            cp = pltpu.make_async_copy(
                ftab_h.at[niv.at[pl.ds(s, 128)]], fbuf, sem)
            cp.start()
            cp.wait()
            pltpu.sync_copy(fbuf, fs_h.at[pl.ds(base + s, 128)])
            return 0

        lax.fori_loop(0, njc, g_body, 0)

    return k(cx, cy, ftab)


def _mlp_body(fs_ref, qm_ref,
              cw1, cb1, cw2, cb2, lw1, lb1, lw2, lb2,
              hw1, hb1, hw2, hb2, hw3, hb3,
              pred_ref, diff_ref):
    ft = fs_ref[...].T
    qm = qm_ref[...]
    inp = jnp.concatenate([ft[0:64], qm[2:4]], axis=0)

    h = jnp.maximum(jnp.dot(cw1[...], inp, preferred_element_type=F32)
                    + cb1[...], 0.0)
    d = jnp.dot(cw2[...], h, preferred_element_type=F32) + cb2[...]
    m = jnp.max(d, axis=0, keepdims=True)
    e = jnp.exp(d - m)
    diff = e / jnp.sum(e, axis=0, keepdims=True)

    hl = jnp.maximum(jnp.dot(lw1[...], inp, preferred_element_type=F32)
                     + lb1[...], 0.0)
    light = jnp.dot(lw2[...], hl, preferred_element_type=F32) + lb2[...]

    hh = jnp.maximum(jnp.dot(hw1[...], inp, preferred_element_type=F32)
                     + hb1[...], 0.0)
    hh = jnp.maximum(jnp.dot(hw2[...], hh, preferred_element_type=F32)
                     + hb2[...], 0.0)
    heavy = jnp.dot(hw3[...], hh, preferred_element_type=F32) + hb3[...]

    cy = qm[0:1]
    cx = qm[1:2]
    fx = (cx + 1.0) * 64.0 - 0.5
    fy = (cy + 1.0) * 64.0 - 0.5
    xi = jnp.minimum(jnp.maximum(fx + 0.5, 0.0).astype(I32), 127)
    yi = jnp.minimum(jnp.maximum(fy + 0.5, 0.0).astype(I32), 127)
    x0 = jnp.minimum(jnp.maximum(fx, 0.0).astype(I32), 127)
    y0 = jnp.minimum(jnp.maximum(fy, 0.0).astype(I32), 127)
    wx = fx - jnp.floor(fx)
    wy = fy - jnp.floor(fy)

    def tap(oy, ox):
        c0 = 64 + ((oy + 1) * 3 + (ox + 1)) * 3
        return ft[c0:c0 + 3]

    ym = y0 < yi
    yp = jnp.logical_and(y0 == yi, y0 < 127)
    xm = x0 < xi
    xp = jnp.logical_and(x0 == xi, x0 < 127)

    def pick(rc, t1, t0):
        return jnp.where(rc, t1, t0)

    v00 = pick(ym, pick(xm, tap(-1, -1), tap(-1, 0)),
               pick(xm, tap(0, -1), tap(0, 0)))
    v01 = pick(ym, pick(xp, tap(-1, 1), tap(-1, 0)),
               pick(xp, tap(0, 1), tap(0, 0)))
    v10 = pick(yp, pick(xm, tap(1, -1), tap(1, 0)),
               pick(xm, tap(0, -1), tap(0, 0)))
    v11 = pick(yp, pick(xp, tap(1, 1), tap(1, 0)),
               pick(xp, tap(0, 1), tap(0, 0)))

    bil = (v00 * (1.0 - wx) * (1.0 - wy) + v01 * wx * (1.0 - wy)
           + v10 * (1.0 - wx) * wy + v11 * wx * wy)

    pred_ref[...] = diff[0:1] * light + diff[1:2] * heavy + bil
    diff_ref[...] = diff


def _mlp_call(fs, qmeta, wts, N, bk):
    grid = (N // bk,)
    row = lambda i: (i, 0)
    col = lambda i: (0, i)
    cst = lambda i: (0, 0)
    w_specs = [pl.BlockSpec(w.shape, cst) for w in wts]
    return pl.pallas_call(
        _mlp_body,
        grid=grid,
        in_specs=[
            pl.BlockSpec((bk, 128), row),
            pl.BlockSpec((4, bk), col),
        ] + w_specs,
        out_specs=[pl.BlockSpec((3, bk), col), pl.BlockSpec((2, bk), col)],
        out_shape=[jax.ShapeDtypeStruct((3, N), F32),
                   jax.ShapeDtypeStruct((2, N), F32)],
    )(fs, qmeta, *wts)


def kernel(lr, coord, cell, enc_w, enc_b, cls_w1, cls_b1, cls_w2, cls_b2,
           lt_w1, lt_b1, lt_w2, lt_b2, hv_w1, hv_b1, hv_w2, hv_b2,
           hv_w3, hv_b3):
    B, C, H, W = lr.shape
    _, Hq, Wq, _ = coord.shape
    D = enc_w.shape[0]
    N = B * Hq * Wq
    q_per_b = Hq * Wq

    lrh = jnp.transpose(lr, (0, 2, 3, 1))
    xpad = jnp.pad(lrh, ((0, 0), (1, 1), (1, 1), (0, 0)))
    xflat = xpad.reshape(B, 130 * 130, 3)
    xflat = jnp.pad(xflat, ((0, 0), (0, 4), (0, 0)))
    nstrip, s_out = 8, 2080
    s_in = 2344
    xstrips = jnp.stack(
        [xflat[:, s * s_out: s * s_out + s_in] for s in range(nstrip)], axis=1)
    wr = jnp.transpose(enc_w, (2, 3, 1, 0)).reshape(27, D)
    f_stride = nstrip * s_out

    ftab = _conv_table(xstrips, wr, enc_b.reshape(1, D),
                       B, nstrip, s_in, s_out, D)
    ftab = ftab.reshape(B * f_stride, 128)

    cx = coord[..., 1].reshape(N)
    cy = coord[..., 0].reshape(N)

    fs = _sc_gather(cx, cy, ftab, N, f_stride, q_per_b)

    qmeta = jnp.stack([cy, cx,
                       cell[..., 0].reshape(N), cell[..., 1].reshape(N)],
                      axis=0)
    wts = (
        cls_w1, cls_b1.reshape(-1, 1), cls_w2, cls_b2.reshape(-1, 1),
        lt_w1, lt_b1.reshape(-1, 1), lt_w2, lt_b2.reshape(-1, 1),
        hv_w1, hv_b1.reshape(-1, 1), hv_w2, hv_b2.reshape(-1, 1),
        hv_w3, hv_b3.reshape(-1, 1),
    )
    predt, difft = _mlp_call(fs, qmeta, wts, N, bk=2048)

    pred = predt.reshape(3, B, Hq, Wq).transpose(1, 0, 2, 3)
    diff = difft.reshape(2, B, Hq, Wq).transpose(1, 0, 2, 3)
    return (pred, diff)


# speedup vs baseline: 55.7617x; 1.0948x over previous
import functools

import jax
import jax.numpy as jnp
from jax import lax
from jax.experimental import pallas as pl
from jax.experimental.pallas import tpu as pltpu
from jax.experimental.pallas import tpu_sc as plsc

F32 = jnp.float32
BF16 = jnp.bfloat16
I32 = jnp.int32

NC, NS, L = 2, 16, 16
NW = NC * NS


def _conv_body(x_ref, w_ref, b_ref, o_ref):
    s_out = o_ref.shape[0]
    cols = [x_ref[pl.ds(dy * 130 + dx, s_out), :]
            for dy in range(3) for dx in range(3)]
    x = jnp.concatenate(cols, axis=1)
    y = jnp.dot(x, w_ref[...], preferred_element_type=F32) + b_ref[...]
    y = jnp.maximum(y, 0.0)
    o_ref[...] = jnp.concatenate([y, x, jnp.zeros((s_out, 37), F32)], axis=1)


def _conv_table(xstrips, wr, b, B, nstrip, s_in, s_out, D):
    return pl.pallas_call(
        _conv_body,
        grid=(B, nstrip),
        in_specs=[
            pl.BlockSpec((None, None, s_in, 3), lambda b_, s_: (b_, s_, 0, 0)),
            pl.BlockSpec((27, D), lambda b_, s_: (0, 0)),
            pl.BlockSpec((1, D), lambda b_, s_: (0, 0)),
        ],
        out_specs=pl.BlockSpec((None, s_out, 128), lambda b_, s_: (b_, s_, 0)),
        out_shape=jax.ShapeDtypeStruct((B, nstrip * s_out, 128), F32),
    )(xstrips, wr, b)


def _sc_gather(cx, cy, ftab, N, f_stride, q_per_b):
    chunk = N // NW
    nsub = chunk // L
    njc = chunk // 128

    mesh = plsc.VectorSubcoreMesh(core_axis_name="c", subcore_axis_name="s",
                                  num_cores=NC, num_subcores=NS)

    @functools.partial(
        pl.kernel, mesh=mesh,
        out_type=jax.ShapeDtypeStruct((N, 128), F32),
        scratch_types=[
            pltpu.VMEM((chunk,), F32),
            pltpu.VMEM((chunk,), F32),
            pltpu.VMEM((chunk,), I32),
            pltpu.VMEM((2, 128, 128), F32),
            pltpu.SemaphoreType.DMA,
            pltpu.SemaphoreType.DMA,
        ],
    )
    def k(cx_h, cy_h, ftab_h, fs_h, cxv, cyv, niv, fbuf, sema, semb):
        wid = lax.axis_index("s") * NC + lax.axis_index("c")
        base = wid * chunk
        pltpu.sync_copy(cx_h.at[pl.ds(base, chunk)], cxv)
        pltpu.sync_copy(cy_h.at[pl.ds(base, chunk)], cyv)
        b = base // q_per_b
        f_off = b * f_stride

        def idx_body(i, _):
            s = i * L
            cx16 = cxv[pl.ds(s, L)]
            cy16 = cyv[pl.ds(s, L)]
            fx = (cx16 + 1.0) * 64.0 - 0.5
            fy = (cy16 + 1.0) * 64.0 - 0.5
            gx = jnp.maximum(fx + 0.5, 0.0)
            gy = jnp.maximum(fy + 0.5, 0.0)
            xi = jnp.minimum(gx.astype(I32), 127)
            yi = jnp.minimum(gy.astype(I32), 127)
            niv[pl.ds(s, L)] = f_off + yi * 130 + xi
            return 0

        lax.fori_loop(0, nsub, idx_body, 0)

        def start_g(j, par, sm):
            pltpu.make_async_copy(
                ftab_h.at[niv.at[pl.ds(j * 128, 128)]],
                fbuf.at[par], sm).start()

        def wait_g(par, sm):
            pltpu.make_async_copy(
                ftab_h.at[niv.at[pl.ds(0, 128)]], fbuf.at[par], sm).wait()

        start_g(0, 0, sema)
        start_g(1, 1, semb)

        def g_body(j2, _):
            for par in range(2):
                j = j2 * 2 + par
                sm = sema if par == 0 else semb
                wait_g(par, sm)
                pltpu.sync_copy(fbuf.at[par],
                                fs_h.at[pl.ds(base + j * 128, 128)])

                @pl.when(j + 2 < njc)
                def _():
                    start_g(j + 2, par, sm)
            return 0

        lax.fori_loop(0, njc // 2, g_body, 0)

    return k(cx, cy, ftab)


def _mlp_body(fs_ref, qm_ref,
              cw1, cb1, cw2, cb2, lw1, lb1, lw2, lb2,
              hw1, hb1, hw2, hb2, hw3, hb3,
              pred_ref, diff_ref):
    ft = fs_ref[...].T
    qm = qm_ref[...]
    inp = jnp.concatenate([ft[0:64], qm[2:4]], axis=0)

    h = jnp.maximum(jnp.dot(cw1[...], inp, preferred_element_type=F32)
                    + cb1[...], 0.0)
    d = jnp.dot(cw2[...], h, preferred_element_type=F32) + cb2[...]
    m = jnp.max(d, axis=0, keepdims=True)
    e = jnp.exp(d - m)
    diff = e / jnp.sum(e, axis=0, keepdims=True)

    hl = jnp.maximum(jnp.dot(lw1[...], inp, preferred_element_type=F32)
                     + lb1[...], 0.0)
    light = jnp.dot(lw2[...], hl, preferred_element_type=F32) + lb2[...]

    hh = jnp.maximum(jnp.dot(hw1[...], inp, preferred_element_type=F32)
                     + hb1[...], 0.0)
    hh = jnp.maximum(jnp.dot(hw2[...], hh, preferred_element_type=F32)
                     + hb2[...], 0.0)
    heavy = jnp.dot(hw3[...], hh, preferred_element_type=F32) + hb3[...]

    cy = qm[0:1]
    cx = qm[1:2]
    fx = (cx + 1.0) * 64.0 - 0.5
    fy = (cy + 1.0) * 64.0 - 0.5
    xi = jnp.minimum(jnp.maximum(fx + 0.5, 0.0).astype(I32), 127)
    yi = jnp.minimum(jnp.maximum(fy + 0.5, 0.0).astype(I32), 127)
    x0 = jnp.minimum(jnp.maximum(fx, 0.0).astype(I32), 127)
    y0 = jnp.minimum(jnp.maximum(fy, 0.0).astype(I32), 127)
    wx = fx - jnp.floor(fx)
    wy = fy - jnp.floor(fy)

    def tap(oy, ox):
        c0 = 64 + ((oy + 1) * 3 + (ox + 1)) * 3
        return ft[c0:c0 + 3]

    ym = y0 < yi
    yp = jnp.logical_and(y0 == yi, y0 < 127)
    xm = x0 < xi
    xp = jnp.logical_and(x0 == xi, x0 < 127)

    def pick(rc, t1, t0):
        return jnp.where(rc, t1, t0)

    v00 = pick(ym, pick(xm, tap(-1, -1), tap(-1, 0)),
               pick(xm, tap(0, -1), tap(0, 0)))
    v01 = pick(ym, pick(xp, tap(-1, 1), tap(-1, 0)),
               pick(xp, tap(0, 1), tap(0, 0)))
    v10 = pick(yp, pick(xm, tap(1, -1), tap(1, 0)),
               pick(xm, tap(0, -1), tap(0, 0)))
    v11 = pick(yp, pick(xp, tap(1, 1), tap(1, 0)),
               pick(xp, tap(0, 1), tap(0, 0)))

    bil = (v00 * (1.0 - wx) * (1.0 - wy) + v01 * wx * (1.0 - wy)
           + v10 * (1.0 - wx) * wy + v11 * wx * wy)

    pred_ref[...] = diff[0:1] * light + diff[1:2] * heavy + bil
    diff_ref[...] = diff


def _mlp_call(fs, qmeta, wts, N, bk):
    grid = (N // bk,)
    row = lambda i: (i, 0)
    col = lambda i: (0, i)
    cst = lambda i: (0, 0)
    w_specs = [pl.BlockSpec(w.shape, cst) for w in wts]
    return pl.pallas_call(
        _mlp_body,
        grid=grid,
        in_specs=[
            pl.BlockSpec((bk, 128), row),
            pl.BlockSpec((4, bk), col),
        ] + w_specs,
        out_specs=[pl.BlockSpec((3, bk), col), pl.BlockSpec((2, bk), col)],
        out_shape=[jax.ShapeDtypeStruct((3, N), F32),
                   jax.ShapeDtypeStruct((2, N), F32)],
    )(fs, qmeta, *wts)


def kernel(lr, coord, cell, enc_w, enc_b, cls_w1, cls_b1, cls_w2, cls_b2,
           lt_w1, lt_b1, lt_w2, lt_b2, hv_w1, hv_b1, hv_w2, hv_b2,
           hv_w3, hv_b3):
    B, C, H, W = lr.shape
    _, Hq, Wq, _ = coord.shape
    D = enc_w.shape[0]
    N = B * Hq * Wq
    q_per_b = Hq * Wq

    lrh = jnp.transpose(lr, (0, 2, 3, 1))
    xpad = jnp.pad(lrh, ((0, 0), (1, 1), (1, 1), (0, 0)))
    xflat = xpad.reshape(B, 130 * 130, 3)
    xflat = jnp.pad(xflat, ((0, 0), (0, 4), (0, 0)))
    nstrip, s_out = 8, 2080
    s_in = 2344
    xstrips = jnp.stack(
        [xflat[:, s * s_out: s * s_out + s_in] for s in range(nstrip)], axis=1)
    wr = jnp.transpose(enc_w, (2, 3, 1, 0)).reshape(27, D)
    f_stride = nstrip * s_out

    ftab = _conv_table(xstrips, wr, enc_b.reshape(1, D),
                       B, nstrip, s_in, s_out, D)
    ftab = ftab.reshape(B * f_stride, 128)

    cx = coord[..., 1].reshape(N)
    cy = coord[..., 0].reshape(N)

    fs = _sc_gather(cx, cy, ftab, N, f_stride, q_per_b)

    qmeta = jnp.stack([cy, cx,
                       cell[..., 0].reshape(N), cell[..., 1].reshape(N)],
                      axis=0)
    wts = (
        cls_w1, cls_b1.reshape(-1, 1), cls_w2, cls_b2.reshape(-1, 1),
        lt_w1, lt_b1.reshape(-1, 1), lt_w2, lt_b2.reshape(-1, 1),
        hv_w1, hv_b1.reshape(-1, 1), hv_w2, hv_b2.reshape(-1, 1),
        hv_w3, hv_b3.reshape(-1, 1),
    )
    predt, difft = _mlp_call(fs, qmeta, wts, N, bk=2048)

    pred = predt.reshape(3, B, Hq, Wq).transpose(1, 0, 2, 3)
    diff = difft.reshape(2, B, Hq, Wq).transpose(1, 0, 2, 3)
    return (pred, diff)


# speedup vs baseline: 57.8610x; 1.0376x over previous
import functools

import jax
import jax.numpy as jnp
from jax import lax
from jax.experimental import pallas as pl
from jax.experimental.pallas import tpu as pltpu
from jax.experimental.pallas import tpu_sc as plsc

F32 = jnp.float32
BF16 = jnp.bfloat16
I32 = jnp.int32

NC, NS, L = 2, 16, 16
NW = NC * NS


def _conv_body(x_ref, w_ref, b_ref, o_ref):
    s_out = o_ref.shape[0]
    cols = [x_ref[pl.ds(dy * 130 + dx, s_out), :]
            for dy in range(3) for dx in range(3)]
    x = jnp.concatenate(cols, axis=1)
    y = jnp.dot(x, w_ref[...], preferred_element_type=F32) + b_ref[...]
    y = jnp.maximum(y, 0.0)
    o_ref[...] = jnp.concatenate([y, x, jnp.zeros((s_out, 37), F32)], axis=1)


def _conv_table(xstrips, wr, b, B, nstrip, s_in, s_out, D):
    return pl.pallas_call(
        _conv_body,
        grid=(B, nstrip),
        in_specs=[
            pl.BlockSpec((None, None, s_in, 3), lambda b_, s_: (b_, s_, 0, 0)),
            pl.BlockSpec((27, D), lambda b_, s_: (0, 0)),
            pl.BlockSpec((1, D), lambda b_, s_: (0, 0)),
        ],
        out_specs=pl.BlockSpec((None, s_out, 128), lambda b_, s_: (b_, s_, 0)),
        out_shape=jax.ShapeDtypeStruct((B, nstrip * s_out, 128), F32),
    )(xstrips, wr, b)


def _sc_gather(cx, cy, ftab, N, f_stride, q_per_b, goff):
    chunk = N // NW
    nsub = chunk // L
    njc = chunk // 128

    mesh = plsc.VectorSubcoreMesh(core_axis_name="c", subcore_axis_name="s",
                                  num_cores=NC, num_subcores=NS)

    @functools.partial(
        pl.kernel, mesh=mesh,
        out_type=jax.ShapeDtypeStruct((N, 128), F32),
        scratch_types=[
            pltpu.VMEM((chunk,), F32),
            pltpu.VMEM((chunk,), F32),
            pltpu.VMEM((chunk,), I32),
            pltpu.VMEM((2, 128, 128), F32),
            pltpu.SemaphoreType.DMA,
            pltpu.SemaphoreType.DMA,
        ],
    )
    def k(cx_h, cy_h, ftab_h, fs_h, cxv, cyv, niv, fbuf, sema, semb):
        wid = lax.axis_index("s") * NC + lax.axis_index("c")
        base = wid * chunk
        pltpu.sync_copy(cx_h.at[pl.ds(base, chunk)], cxv)
        pltpu.sync_copy(cy_h.at[pl.ds(base, chunk)], cyv)
        b = (goff + base) // q_per_b
        f_off = b * f_stride

        def idx_body(i, _):
            s = i * L
            cx16 = cxv[pl.ds(s, L)]
            cy16 = cyv[pl.ds(s, L)]
            fx = (cx16 + 1.0) * 64.0 - 0.5
            fy = (cy16 + 1.0) * 64.0 - 0.5
            gx = jnp.maximum(fx + 0.5, 0.0)
            gy = jnp.maximum(fy + 0.5, 0.0)
            xi = jnp.minimum(gx.astype(I32), 127)
            yi = jnp.minimum(gy.astype(I32), 127)
            niv[pl.ds(s, L)] = f_off + yi * 130 + xi
            return 0

        lax.fori_loop(0, nsub, idx_body, 0)

        def start_g(j, par, sm):
            pltpu.make_async_copy(
                ftab_h.at[niv.at[pl.ds(j * 128, 128)]],
                fbuf.at[par], sm).start()

        def wait_g(par, sm):
            pltpu.make_async_copy(
                ftab_h.at[niv.at[pl.ds(0, 128)]], fbuf.at[par], sm).wait()

        start_g(0, 0, sema)
        start_g(1, 1, semb)

        def g_body(j2, _):
            for par in range(2):
                j = j2 * 2 + par
                sm = sema if par == 0 else semb
                wait_g(par, sm)
                pltpu.sync_copy(fbuf.at[par],
                                fs_h.at[pl.ds(base + j * 128, 128)])

                @pl.when(j + 2 < njc)
                def _():
                    start_g(j + 2, par, sm)
            return 0

        lax.fori_loop(0, njc // 2, g_body, 0)

    return k(cx, cy, ftab)


def _mlp_body(fs_ref, qm_ref,
              cw1, cb1, cw2, cb2, lw1, lb1, lw2, lb2,
              hw1, hb1, hw2, hb2, hw3, hb3,
              pred_ref, diff_ref):
    ft = fs_ref[...].T
    qm = qm_ref[...]
    inp = jnp.concatenate([ft[0:64], qm[2:4]], axis=0)

    h = jnp.maximum(jnp.dot(cw1[...], inp, preferred_element_type=F32)
                    + cb1[...], 0.0)
    d = jnp.dot(cw2[...], h, preferred_element_type=F32) + cb2[...]
    m = jnp.max(d, axis=0, keepdims=True)
    e = jnp.exp(d - m)
    diff = e / jnp.sum(e, axis=0, keepdims=True)

    hl = jnp.maximum(jnp.dot(lw1[...], inp, preferred_element_type=F32)
                     + lb1[...], 0.0)
    light = jnp.dot(lw2[...], hl, preferred_element_type=F32) + lb2[...]

    hh = jnp.maximum(jnp.dot(hw1[...], inp, preferred_element_type=F32)
                     + hb1[...], 0.0)
    hh = jnp.maximum(jnp.dot(hw2[...], hh, preferred_element_type=F32)
                     + hb2[...], 0.0)
    heavy = jnp.dot(hw3[...], hh, preferred_element_type=F32) + hb3[...]

    cy = qm[0:1]
    cx = qm[1:2]
    fx = (cx + 1.0) * 64.0 - 0.5
    fy = (cy + 1.0) * 64.0 - 0.5
    xi = jnp.minimum(jnp.maximum(fx + 0.5, 0.0).astype(I32), 127)
    yi = jnp.minimum(jnp.maximum(fy + 0.5, 0.0).astype(I32), 127)
    x0 = jnp.minimum(jnp.maximum(fx, 0.0).astype(I32), 127)
    y0 = jnp.minimum(jnp.maximum(fy, 0.0).astype(I32), 127)
    wx = fx - jnp.floor(fx)
    wy = fy - jnp.floor(fy)

    def tap(oy, ox):
        c0 = 64 + ((oy + 1) * 3 + (ox + 1)) * 3
        return ft[c0:c0 + 3]

    ym = y0 < yi
    yp = jnp.logical_and(y0 == yi, y0 < 127)
    xm = x0 < xi
    xp = jnp.logical_and(x0 == xi, x0 < 127)

    def pick(rc, t1, t0):
        return jnp.where(rc, t1, t0)

    v00 = pick(ym, pick(xm, tap(-1, -1), tap(-1, 0)),
               pick(xm, tap(0, -1), tap(0, 0)))
    v01 = pick(ym, pick(xp, tap(-1, 1), tap(-1, 0)),
               pick(xp, tap(0, 1), tap(0, 0)))
    v10 = pick(yp, pick(xm, tap(1, -1), tap(1, 0)),
               pick(xm, tap(0, -1), tap(0, 0)))
    v11 = pick(yp, pick(xp, tap(1, 1), tap(1, 0)),
               pick(xp, tap(0, 1), tap(0, 0)))

    bil = (v00 * (1.0 - wx) * (1.0 - wy) + v01 * wx * (1.0 - wy)
           + v10 * (1.0 - wx) * wy + v11 * wx * wy)

    pred_ref[...] = diff[0:1] * light + diff[1:2] * heavy + bil
    diff_ref[...] = diff


def _mlp_call(fs, qmeta, wts, N, bk):
    grid = (N // bk,)
    row = lambda i: (i, 0)
    col = lambda i: (0, i)
    cst = lambda i: (0, 0)
    w_specs = [pl.BlockSpec(w.shape, cst) for w in wts]
    return pl.pallas_call(
        _mlp_body,
        grid=grid,
        in_specs=[
            pl.BlockSpec((bk, 128), row),
            pl.BlockSpec((4, bk), col),
        ] + w_specs,
        out_specs=[pl.BlockSpec((3, bk), col), pl.BlockSpec((2, bk), col)],
        out_shape=[jax.ShapeDtypeStruct((3, N), F32),
                   jax.ShapeDtypeStruct((2, N), F32)],
    )(fs, qmeta, *wts)


def kernel(lr, coord, cell, enc_w, enc_b, cls_w1, cls_b1, cls_w2, cls_b2,
           lt_w1, lt_b1, lt_w2, lt_b2, hv_w1, hv_b1, hv_w2, hv_b2,
           hv_w3, hv_b3):
    B, C, H, W = lr.shape
    _, Hq, Wq, _ = coord.shape
    D = enc_w.shape[0]
    N = B * Hq * Wq
    q_per_b = Hq * Wq

    lrh = jnp.transpose(lr, (0, 2, 3, 1))
    xpad = jnp.pad(lrh, ((0, 0), (1, 1), (1, 1), (0, 0)))
    xflat = xpad.reshape(B, 130 * 130, 3)
    xflat = jnp.pad(xflat, ((0, 0), (0, 4), (0, 0)))
    nstrip, s_out = 8, 2080
    s_in = 2344
    xstrips = jnp.stack(
        [xflat[:, s * s_out: s * s_out + s_in] for s in range(nstrip)], axis=1)
    wr = jnp.transpose(enc_w, (2, 3, 1, 0)).reshape(27, D)
    f_stride = nstrip * s_out

    ftab = _conv_table(xstrips, wr, enc_b.reshape(1, D),
                       B, nstrip, s_in, s_out, D)
    ftab = ftab.reshape(B * f_stride, 128)

    cx = coord[..., 1].reshape(N)
    cy = coord[..., 0].reshape(N)

    qmeta = jnp.stack([cy, cx,
                       cell[..., 0].reshape(N), cell[..., 1].reshape(N)],
                      axis=0)
    wts = (
        cls_w1, cls_b1.reshape(-1, 1), cls_w2, cls_b2.reshape(-1, 1),
        lt_w1, lt_b1.reshape(-1, 1), lt_w2, lt_b2.reshape(-1, 1),
        hv_w1, hv_b1.reshape(-1, 1), hv_w2, hv_b2.reshape(-1, 1),
        hv_w3, hv_b3.reshape(-1, 1),
    )

    K = 2
    nk = N // K
    preds, diffs = [], []
    for k in range(K):
        sl = slice(k * nk, (k + 1) * nk)
        fs_k = _sc_gather(cx[sl], cy[sl], ftab, nk, f_stride, q_per_b, k * nk)
        pt, dt = _mlp_call(fs_k, qmeta[:, sl], wts, nk, bk=2048)
        preds.append(pt)
        diffs.append(dt)
    predt = jnp.concatenate(preds, axis=1)
    difft = jnp.concatenate(diffs, axis=1)

    pred = predt.reshape(3, B, Hq, Wq).transpose(1, 0, 2, 3)
    diff = difft.reshape(2, B, Hq, Wq).transpose(1, 0, 2, 3)
    return (pred, diff)


# speedup vs baseline: 62.4591x; 1.0795x over previous
import functools

import jax
import jax.numpy as jnp
from jax import lax
from jax.experimental import pallas as pl
from jax.experimental.pallas import tpu as pltpu
from jax.experimental.pallas import tpu_sc as plsc

F32 = jnp.float32
BF16 = jnp.bfloat16
I32 = jnp.int32

NC, NS, L = 2, 16, 16
NW = NC * NS


def _conv_body(x_ref, w_ref, b_ref, o_ref):
    s_out = o_ref.shape[0]
    xs = x_ref[...].T
    cols = [xs[dy * 130 + dx: dy * 130 + dx + s_out, :]
            for dy in range(3) for dx in range(3)]
    x = jnp.concatenate(cols, axis=1)
    y = jnp.dot(x, w_ref[...], preferred_element_type=F32) + b_ref[...]
    y = jnp.maximum(y, 0.0)
    o_ref[...] = jnp.concatenate([y, x, jnp.zeros((s_out, 37), F32)], axis=1)


def _conv_table(xstrips, wr, b, B, nstrip, s_in, s_out, D):
    return pl.pallas_call(
        _conv_body,
        grid=(B, nstrip),
        in_specs=[
            pl.BlockSpec((None, None, 3, s_in), lambda b_, s_: (b_, s_, 0, 0)),
            pl.BlockSpec((27, D), lambda b_, s_: (0, 0)),
            pl.BlockSpec((1, D), lambda b_, s_: (0, 0)),
        ],
        out_specs=pl.BlockSpec((None, s_out, 128), lambda b_, s_: (b_, s_, 0)),
        out_shape=jax.ShapeDtypeStruct((B, nstrip * s_out, 128), F32),
    )(xstrips, wr, b)


def _sc_gather(cx, cy, ftab, N, f_stride, q_per_b, goff):
    chunk = N // NW
    nsub = chunk // L
    njc = chunk // 128

    mesh = plsc.VectorSubcoreMesh(core_axis_name="c", subcore_axis_name="s",
                                  num_cores=NC, num_subcores=NS)

    @functools.partial(
        pl.kernel, mesh=mesh,
        out_type=jax.ShapeDtypeStruct((N, 128), F32),
        scratch_types=[
            pltpu.VMEM((chunk,), F32),
            pltpu.VMEM((chunk,), F32),
            pltpu.VMEM((chunk,), I32),
            pltpu.VMEM((2, 128, 128), F32),
            pltpu.SemaphoreType.DMA,
            pltpu.SemaphoreType.DMA,
        ],
    )
    def k(cx_h, cy_h, ftab_h, fs_h, cxv, cyv, niv, fbuf, sema, semb):
        wid = lax.axis_index("s") * NC + lax.axis_index("c")
        base = wid * chunk
        pltpu.sync_copy(cx_h.at[pl.ds(base, chunk)], cxv)
        pltpu.sync_copy(cy_h.at[pl.ds(base, chunk)], cyv)
        b = (goff + base) // q_per_b
        f_off = b * f_stride

        def idx_body(i, _):
            s = i * L
            cx16 = cxv[pl.ds(s, L)]
            cy16 = cyv[pl.ds(s, L)]
            fx = (cx16 + 1.0) * 64.0 - 0.5
            fy = (cy16 + 1.0) * 64.0 - 0.5
            gx = jnp.maximum(fx + 0.5, 0.0)
            gy = jnp.maximum(fy + 0.5, 0.0)
            xi = jnp.minimum(gx.astype(I32), 127)
            yi = jnp.minimum(gy.astype(I32), 127)
            niv[pl.ds(s, L)] = f_off + yi * 130 + xi
            return 0

        lax.fori_loop(0, nsub, idx_body, 0)

        def start_g(j, par, sm):
            pltpu.make_async_copy(
                ftab_h.at[niv.at[pl.ds(j * 128, 128)]],
                fbuf.at[par], sm).start()

        def wait_g(par, sm):
            pltpu.make_async_copy(
                ftab_h.at[niv.at[pl.ds(0, 128)]], fbuf.at[par], sm).wait()

        start_g(0, 0, sema)
        start_g(1, 1, semb)

        def g_body(j2, _):
            for par in range(2):
                j = j2 * 2 + par
                sm = sema if par == 0 else semb
                wait_g(par, sm)
                pltpu.sync_copy(fbuf.at[par],
                                fs_h.at[pl.ds(base + j * 128, 128)])

                @pl.when(j + 2 < njc)
                def _():
                    start_g(j + 2, par, sm)
            return 0

        lax.fori_loop(0, njc // 2, g_body, 0)

    return k(cx, cy, ftab)


def _mlp_body(fs_ref, qm_ref,
              cw1, cb1, cw2, cb2, lw1, lb1, lw2, lb2,
              hw1, hb1, hw2, hb2, hw3, hb3,
              pred_ref, diff_ref):
    ft = fs_ref[...].T
    qm = qm_ref[...]
    inp = jnp.concatenate([ft[0:64], qm[2:4]], axis=0)

    h = jnp.maximum(jnp.dot(cw1[...], inp, preferred_element_type=F32)
                    + cb1[...], 0.0)
    d = jnp.dot(cw2[...], h, preferred_element_type=F32) + cb2[...]
    m = jnp.max(d, axis=0, keepdims=True)
    e = jnp.exp(d - m)
    diff = e / jnp.sum(e, axis=0, keepdims=True)

    hl = jnp.maximum(jnp.dot(lw1[...], inp, preferred_element_type=F32)
                     + lb1[...], 0.0)
    light = jnp.dot(lw2[...], hl, preferred_element_type=F32) + lb2[...]

    hh = jnp.maximum(jnp.dot(hw1[...], inp, preferred_element_type=F32)
                     + hb1[...], 0.0)
    hh = jnp.maximum(jnp.dot(hw2[...], hh, preferred_element_type=F32)
                     + hb2[...], 0.0)
    heavy = jnp.dot(hw3[...], hh, preferred_element_type=F32) + hb3[...]

    cy = qm[0:1]
    cx = qm[1:2]
    fx = (cx + 1.0) * 64.0 - 0.5
    fy = (cy + 1.0) * 64.0 - 0.5
    xi = jnp.minimum(jnp.maximum(fx + 0.5, 0.0).astype(I32), 127)
    yi = jnp.minimum(jnp.maximum(fy + 0.5, 0.0).astype(I32), 127)
    x0 = jnp.minimum(jnp.maximum(fx, 0.0).astype(I32), 127)
    y0 = jnp.minimum(jnp.maximum(fy, 0.0).astype(I32), 127)
    wx = fx - jnp.floor(fx)
    wy = fy - jnp.floor(fy)

    def tap(oy, ox):
        c0 = 64 + ((oy + 1) * 3 + (ox + 1)) * 3
        return ft[c0:c0 + 3]

    ym = y0 < yi
    yp = jnp.logical_and(y0 == yi, y0 < 127)
    xm = x0 < xi
    xp = jnp.logical_and(x0 == xi, x0 < 127)

    def pick(rc, t1, t0):
        return jnp.where(rc, t1, t0)

    v00 = pick(ym, pick(xm, tap(-1, -1), tap(-1, 0)),
               pick(xm, tap(0, -1), tap(0, 0)))
    v01 = pick(ym, pick(xp, tap(-1, 1), tap(-1, 0)),
               pick(xp, tap(0, 1), tap(0, 0)))
    v10 = pick(yp, pick(xm, tap(1, -1), tap(1, 0)),
               pick(xm, tap(0, -1), tap(0, 0)))
    v11 = pick(yp, pick(xp, tap(1, 1), tap(1, 0)),
               pick(xp, tap(0, 1), tap(0, 0)))

    bil = (v00 * (1.0 - wx) * (1.0 - wy) + v01 * wx * (1.0 - wy)
           + v10 * (1.0 - wx) * wy + v11 * wx * wy)

    pred_ref[...] = diff[0:1] * light + diff[1:2] * heavy + bil
    diff_ref[...] = diff


def _mlp_call(fs, qmeta, wts, N, bk):
    grid = (N // bk,)
    row = lambda i: (i, 0)
    col = lambda i: (0, i)
    cst = lambda i: (0, 0)
    w_specs = [pl.BlockSpec(w.shape, cst) for w in wts]
    return pl.pallas_call(
        _mlp_body,
        grid=grid,
        in_specs=[
            pl.BlockSpec((bk, 128), row),
            pl.BlockSpec((4, bk), col),
        ] + w_specs,
        out_specs=[pl.BlockSpec((3, bk), col), pl.BlockSpec((2, bk), col)],
        out_shape=[jax.ShapeDtypeStruct((3, N), F32),
                   jax.ShapeDtypeStruct((2, N), F32)],
    )(fs, qmeta, *wts)


def kernel(lr, coord, cell, enc_w, enc_b, cls_w1, cls_b1, cls_w2, cls_b2,
           lt_w1, lt_b1, lt_w2, lt_b2, hv_w1, hv_b1, hv_w2, hv_b2,
           hv_w3, hv_b3):
    B, C, H, W = lr.shape
    _, Hq, Wq, _ = coord.shape
    D = enc_w.shape[0]
    N = B * Hq * Wq
    q_per_b = Hq * Wq

    lrp = jnp.pad(lr, ((0, 0), (0, 0), (1, 1), (1, 1)))
    xflat = lrp.reshape(B, 3, 130 * 130)
    xflat = jnp.pad(xflat, ((0, 0), (0, 0), (0, 4)))
    nstrip, s_out = 8, 2080
    s_in = 2344
    xstrips = jnp.stack(
        [xflat[:, :, s * s_out: s * s_out + s_in] for s in range(nstrip)],
        axis=1)
    wr = jnp.transpose(enc_w, (2, 3, 1, 0)).reshape(27, D)
    f_stride = nstrip * s_out

    ftab = _conv_table(xstrips, wr, enc_b.reshape(1, D),
                       B, nstrip, s_in, s_out, D)
    ftab = ftab.reshape(B * f_stride, 128)

    cx = coord[..., 1].reshape(N)
    cy = coord[..., 0].reshape(N)

    qmeta = jnp.stack([cy, cx,
                       cell[..., 0].reshape(N), cell[..., 1].reshape(N)],
                      axis=0)
    wts = (
        cls_w1, cls_b1.reshape(-1, 1), cls_w2, cls_b2.reshape(-1, 1),
        lt_w1, lt_b1.reshape(-1, 1), lt_w2, lt_b2.reshape(-1, 1),
        hv_w1, hv_b1.reshape(-1, 1), hv_w2, hv_b2.reshape(-1, 1),
        hv_w3, hv_b3.reshape(-1, 1),
    )

    K = 4
    nk = N // K
    preds, diffs = [], []
    for k in range(K):
        sl = slice(k * nk, (k + 1) * nk)
        fs_k = _sc_gather(cx[sl], cy[sl], ftab, nk, f_stride, q_per_b, k * nk)
        pt, dt = _mlp_call(fs_k, qmeta[:, sl], wts, nk, bk=2048)
        preds.append(pt)
        diffs.append(dt)
    predt = jnp.concatenate(preds, axis=1)
    difft = jnp.concatenate(diffs, axis=1)

    pred = predt.reshape(3, B, Hq, Wq).transpose(1, 0, 2, 3)
    diff = difft.reshape(2, B, Hq, Wq).transpose(1, 0, 2, 3)
    return (pred, diff)
